# baseline (speedup 1.0000x reference)
import functools

import jax
import jax.numpy as jnp
from jax import lax
from jax.experimental import pallas as pl
from jax.experimental.pallas import tpu as pltpu
from jax.experimental.pallas import tpu_sc as plsc

_D = 64
_NC = 2
_NS = 16
_NW = _NC * _NS
_CHUNK = 128


def _emb_body(table_hbm, idx_hbm, out_hbm, idx_v, rows_v, sem, *, b_per_w):
    wid = lax.axis_index("s") * _NC + lax.axis_index("c")
    nchunks = b_per_w // _CHUNK

    def body(g, carry):
        base = wid * b_per_w + g * _CHUNK
        pltpu.sync_copy(idx_hbm.at[pl.ds(base, _CHUNK)], idx_v)
        pltpu.async_copy(table_hbm.at[idx_v], rows_v, sem).wait()
        pltpu.sync_copy(rows_v, out_hbm.at[pl.ds(base, _CHUNK)])
        return carry

    lax.fori_loop(0, nchunks, body, 0)


def kernel(tokenized_sentence, table):
    batch, seq = tokenized_sentence.shape
    b_total = batch * seq
    idx = tokenized_sentence.reshape(b_total).astype(jnp.int32)
    b_per_w = b_total // _NW

    mesh = plsc.VectorSubcoreMesh(core_axis_name="c", subcore_axis_name="s")
    k = pl.kernel(
        functools.partial(_emb_body, b_per_w=b_per_w),
        mesh=mesh,
        out_type=jax.ShapeDtypeStruct((b_total, _D), jnp.float32),
        scratch_types=[
            pltpu.VMEM((_CHUNK,), jnp.int32),
            pltpu.VMEM((_CHUNK, _D), jnp.float32),
            pltpu.SemaphoreType.DMA,
        ],
        compiler_params=pltpu.CompilerParams(use_tc_tiling_on_sc=False),
    )
    out = k(table, idx)
    return out.reshape(batch, seq, _D)


# speedup vs baseline: 1.1854x; 1.1854x over previous
import functools

import jax
import jax.numpy as jnp
from jax import lax
from jax.experimental import pallas as pl
from jax.experimental.pallas import tpu as pltpu
from jax.experimental.pallas import tpu_sc as plsc

_D = 64
_NC = 2
_NS = 16
_NW = _NC * _NS
_CHUNK = 512


def _emb_body(table_hbm, idx_hbm, out_hbm,
              idx_v, rows0, rows1, gsem0, gsem1, wsem0, wsem1, *, b_per_w):
    wid = lax.axis_index("s") * _NC + lax.axis_index("c")
    base_w = wid * b_per_w
    nchunks = b_per_w // _CHUNK
    ngroups = nchunks // 2

    pltpu.sync_copy(idx_hbm.at[pl.ds(base_w, b_per_w)], idx_v)

    rows = (rows0, rows1)
    gsem = (gsem0, gsem1)
    wsem = (wsem0, wsem1)

    def gather(i, b):
        return pltpu.make_async_copy(
            table_hbm.at[idx_v.at[pl.ds(i * _CHUNK, _CHUNK)]], rows[b], gsem[b])

    def writeback(i, b):
        return pltpu.make_async_copy(
            rows[b], out_hbm.at[pl.ds(base_w + i * _CHUNK, _CHUNK)], wsem[b])

    gather(0, 0).start()
    gather(1, 1).start()
    gather(0, 0).wait()
    writeback(0, 0).start()
    gather(1, 1).wait()
    writeback(1, 1).start()

    def body(g, carry):
        i = g * 2
        writeback(i - 2, 0).wait()
        gather(i, 0).start()
        writeback(i - 1, 1).wait()
        gather(i + 1, 1).start()
        gather(i, 0).wait()
        writeback(i, 0).start()
        gather(i + 1, 1).wait()
        writeback(i + 1, 1).start()
        return carry

    lax.fori_loop(1, ngroups, body, 0)

    writeback(nchunks - 2, 0).wait()
    writeback(nchunks - 1, 1).wait()


def kernel(tokenized_sentence, table):
    batch, seq = tokenized_sentence.shape
    b_total = batch * seq
    idx = tokenized_sentence.reshape(b_total).astype(jnp.int32)
    b_per_w = b_total // _NW

    mesh = plsc.VectorSubcoreMesh(core_axis_name="c", subcore_axis_name="s")
    k = pl.kernel(
        functools.partial(_emb_body, b_per_w=b_per_w),
        mesh=mesh,
        out_type=jax.ShapeDtypeStruct((b_total, _D), jnp.float32),
        scratch_types=[
            pltpu.VMEM((b_per_w,), jnp.int32),
            pltpu.VMEM((_CHUNK, _D), jnp.float32),
            pltpu.VMEM((_CHUNK, _D), jnp.float32),
            pltpu.SemaphoreType.DMA,
            pltpu.SemaphoreType.DMA,
            pltpu.SemaphoreType.DMA,
            pltpu.SemaphoreType.DMA,
        ],
        compiler_params=pltpu.CompilerParams(use_tc_tiling_on_sc=False),
    )
    out = k(table, idx)
    return out.reshape(batch, seq, _D)
